# Initial kernel scaffold; baseline (speedup 1.0000x reference)
import jax
import jax.numpy as jnp
from jax.experimental import pallas as pl


def kernel(z, pos, edge_index, emb, means, betas, rbf_w, rbf_b, nemb, dist_w, dist_b, comb_w, comb_b):
    raise NotImplementedError("write your pallas kernel here")


# speedup vs baseline: 3.0441x; 3.0441x over previous
import math

import jax
import jax.numpy as jnp
from jax import lax
from jax.experimental import pallas as pl
from jax.experimental.pallas import tpu as pltpu
from jax.experimental.pallas import tpu_sc as plsc

CU = 5.0
CL = 0.0
NC = 2
NS = 16
LANES = 16
NW = NC * NS
RPAD = 64
ECHUNK_A = 1024
ECHUNK_C = 128
NCHUNK_A = 64


def _cdiv(a, b):
    return (a + b - 1) // b


def kernel(z, pos, edge_index, emb, means, betas, rbf_w, rbf_b, nemb,
           dist_w, dist_b, comb_w, comb_b):
    N = z.shape[0]
    E = edge_index.shape[1]
    H = emb.shape[1]
    R = means.shape[0]

    n_pad = _cdiv(N, NW * NCHUNK_A) * NW * NCHUNK_A
    e_pad = _cdiv(E, NW * ECHUNK_A) * NW * ECHUNK_A
    npw = n_pad // NW
    epw = e_pad // NW
    nb = e_pad // ECHUNK_A
    f32 = jnp.float32

    z = z.astype(jnp.int32)
    src = edge_index[0].astype(jnp.int32)
    dst = edge_index[1].astype(jnp.int32)
    z_p = jnp.pad(z, (0, n_pad - N))
    src_p = jnp.pad(src, (0, e_pad - E))
    dst_p = jnp.pad(dst, (0, e_pad - E))
    posf = jnp.pad(pos.astype(f32), ((0, 0), (0, 1))).reshape(-1)

    mesh = plsc.VectorSubcoreMesh(core_axis_name="c", subcore_axis_name="s")

    def body_a(z_hbm, posf_hbm, emb_hbm, src_hbm, dst_hbm,
               ne_hbm, evx_hbm, evy_hbm, evz_hbm,
               zidx_v, nbuf_v, posf_v, sidx_v, didx_v, ex_v, ey_v, ez_v,
               sem):
        c = lax.axis_index("c")
        s = lax.axis_index("s")
        wid = s * NC + c
        base_n = wid * npw

        @pl.loop(0, npw // NCHUNK_A)
        def node_chunk(i):
            b = base_n + i * NCHUNK_A
            pltpu.sync_copy(z_hbm.at[pl.ds(b, NCHUNK_A)], zidx_v)
            pltpu.async_copy(emb_hbm.at[zidx_v], nbuf_v, sem).wait()
            pltpu.sync_copy(nbuf_v, ne_hbm.at[pl.ds(b, NCHUNK_A)])

        pltpu.sync_copy(posf_hbm, posf_v)
        base_e = wid * epw

        @pl.loop(0, epw // ECHUNK_A)
        def edge_chunk(k):
            b = base_e + k * ECHUNK_A
            pltpu.sync_copy(src_hbm.at[pl.ds(b, ECHUNK_A)], sidx_v)
            pltpu.sync_copy(dst_hbm.at[pl.ds(b, ECHUNK_A)], didx_v)

            @pl.loop(0, ECHUNK_A // LANES, unroll=4)
            def g(j):
                o = j * LANES
                si = sidx_v[pl.ds(o, LANES)] * 4
                di = didx_v[pl.ds(o, LANES)] * 4
                ex_v[pl.ds(o, LANES)] = (plsc.load_gather(posf_v, [si]) -
                                         plsc.load_gather(posf_v, [di]))
                ey_v[pl.ds(o, LANES)] = (plsc.load_gather(posf_v, [si + 1]) -
                                         plsc.load_gather(posf_v, [di + 1]))
                ez_v[pl.ds(o, LANES)] = (plsc.load_gather(posf_v, [si + 2]) -
                                         plsc.load_gather(posf_v, [di + 2]))

            pltpu.sync_copy(ex_v, evx_hbm.at[pl.ds(b, ECHUNK_A)])
            pltpu.sync_copy(ey_v, evy_hbm.at[pl.ds(b, ECHUNK_A)])
            pltpu.sync_copy(ez_v, evz_hbm.at[pl.ds(b, ECHUNK_A)])

    kern_a = pl.kernel(
        body_a,
        out_type=[
            jax.ShapeDtypeStruct((n_pad, H), f32),
            jax.ShapeDtypeStruct((e_pad,), f32),
            jax.ShapeDtypeStruct((e_pad,), f32),
            jax.ShapeDtypeStruct((e_pad,), f32),
        ],
        mesh=mesh,
        scratch_types=[
            pltpu.VMEM((NCHUNK_A,), jnp.int32),
            pltpu.VMEM((NCHUNK_A, H), f32),
            pltpu.VMEM((4 * N,), f32),
            pltpu.VMEM((ECHUNK_A,), jnp.int32),
            pltpu.VMEM((ECHUNK_A,), jnp.int32),
            pltpu.VMEM((ECHUNK_A,), f32),
            pltpu.VMEM((ECHUNK_A,), f32),
            pltpu.VMEM((ECHUNK_A,), f32),
            pltpu.SemaphoreType.DMA,
        ],
        compiler_params=pltpu.CompilerParams(needs_layout_passes=False),
    )
    ne, evx, evy, evz = kern_a(z_p, posf, emb.astype(f32), src_p, dst_p)

    meansb = jnp.broadcast_to(
        jnp.pad(means.astype(f32), (0, RPAD - R))[:, None], (RPAD, 128))
    betasb = jnp.broadcast_to(
        jnp.pad(betas.astype(f32), (0, RPAD - R))[:, None], (RPAD, 128))
    rbfa = jnp.concatenate(
        [rbf_w.astype(f32), jnp.zeros((RPAD - 1 - R, H), f32),
         rbf_b.astype(f32)[None, :]], axis=0)
    dista = jnp.concatenate(
        [dist_w.astype(f32), jnp.zeros((RPAD - 1 - R, H), f32),
         dist_b.astype(f32)[None, :]], axis=0)

    def body_b(ex_ref, ey_ref, ez_ref, srcb_ref, dstb_ref,
               means_ref, betas_ref, rbfw_ref, distw_ref,
               ew_ref, enx_ref, eny_ref, enz_ref, attr_ref, w_ref):
        ex = ex_ref[0]
        ey = ey_ref[0]
        ez = ez_ref[0]
        d2 = ex * ex + ey * ey + ez * ez
        d = jnp.sqrt(d2)
        ew_ref[0] = d
        inv = 1.0 / d
        enx_ref[0] = ex * inv
        eny_ref[0] = ey * inv
        enz_ref[0] = ez * inv
        cut = 0.5 * (jnp.cos(d * (math.pi / CU)) + 1.0) * (d < CU).astype(f32)
        neq = (srcb_ref[0] != dstb_ref[0]).astype(f32)
        cn = cut * neq
        mm = means_ref[...]
        bb = betas_ref[...]
        rowid = lax.broadcasted_iota(jnp.int32, (RPAD, 128), 0)
        alpha = 5.0 / (CU - CL)
        for j in range(8):
            dj = d[j:j + 1, :]
            attr = jnp.exp(-bb * (jnp.exp(alpha * (CL - dj)) - mm) ** 2)
            attr = attr * cut[j:j + 1, :]
            attr = jnp.where(rowid < R, attr, 0.0)
            attr = jnp.where(rowid == RPAD - 1, 1.0, attr)
            attr_ref[pl.ds(j * 128, 128), :] = lax.dot_general(
                attr, rbfw_ref[...], (((0,), (0,)), ((), ())),
                preferred_element_type=f32)
            attr2 = attr * cn[j:j + 1, :]
            w_ref[pl.ds(j * 128, 128), :] = lax.dot_general(
                attr2, distw_ref[...], (((0,), (0,)), ((), ())),
                preferred_element_type=f32)

    ev_spec = pl.BlockSpec((1, 8, 128), lambda i: (i, 0, 0))
    par_spec = pl.BlockSpec((RPAD, 128), lambda i: (0, 0))
    eh_spec = pl.BlockSpec((ECHUNK_A, H), lambda i: (i, 0))
    ew3, enx3, eny3, enz3, attr_out, w_mat = pl.pallas_call(
        body_b,
        grid=(nb,),
        in_specs=[ev_spec, ev_spec, ev_spec, ev_spec, ev_spec,
                  par_spec, par_spec, par_spec, par_spec],
        out_specs=[ev_spec, ev_spec, ev_spec, ev_spec, eh_spec, eh_spec],
        out_shape=[
            jax.ShapeDtypeStruct((nb, 8, 128), f32),
            jax.ShapeDtypeStruct((nb, 8, 128), f32),
            jax.ShapeDtypeStruct((nb, 8, 128), f32),
            jax.ShapeDtypeStruct((nb, 8, 128), f32),
            jax.ShapeDtypeStruct((e_pad, H), f32),
            jax.ShapeDtypeStruct((e_pad, H), f32),
        ],
    )(evx.reshape(nb, 8, 128), evy.reshape(nb, 8, 128),
      evz.reshape(nb, 8, 128), src_p.reshape(nb, 8, 128),
      dst_p.reshape(nb, 8, 128), meansb, betasb, rbfa, dista)

    zero_init = jnp.zeros((n_pad, H), f32)
    rpt = n_pad // NS

    def body_c(w_hbm, src_hbm, dst_hbm, z_hbm, nemb_hbm, zero_hbm,
               agg_hbm,
               zv, sidx, didx, zsrc, wbuf, xbuf, nemb_sh, agg_sh, sem):
        c = lax.axis_index("c")
        s = lax.axis_index("s")
        pltpu.sync_copy(z_hbm, zv)
        pltpu.sync_copy(zero_hbm.at[pl.ds(s * rpt, rpt)],
                        agg_sh.at[pl.ds(s * rpt, rpt)])

        @pl.when(s == 0)
        def _():
            pltpu.sync_copy(nemb_hbm, nemb_sh)

        plsc.subcore_barrier()
        base_e = (s * NC + c) * epw

        @pl.loop(0, epw // ECHUNK_C)
        def chunk(k):
            b = base_e + k * ECHUNK_C
            pltpu.sync_copy(src_hbm.at[pl.ds(b, ECHUNK_C)], sidx)
            pltpu.sync_copy(dst_hbm.at[pl.ds(b, ECHUNK_C)], didx)

            @pl.loop(0, ECHUNK_C // LANES, unroll=8)
            def zg(j):
                o = j * LANES
                zsrc[pl.ds(o, LANES)] = plsc.load_gather(
                    zv, [sidx[pl.ds(o, LANES)]])

            pltpu.async_copy(nemb_sh.at[zsrc], xbuf, sem).wait()
            pltpu.sync_copy(w_hbm.at[pl.ds(b, ECHUNK_C)], wbuf)

            @pl.loop(0, ECHUNK_C, unroll=2)
            def mul(r):
                for h in range(H // LANES):
                    o = h * LANES
                    wbuf[r, pl.ds(o, LANES)] = (wbuf[r, pl.ds(o, LANES)] *
                                                xbuf[r, pl.ds(o, LANES)])

            pltpu.sync_copy(wbuf, agg_sh.at[didx], add=True)

        plsc.subcore_barrier()
        pltpu.sync_copy(agg_sh.at[pl.ds(s * rpt, rpt)],
                        agg_hbm.at[c, pl.ds(s * rpt, rpt)])

    kern_c = pl.kernel(
        body_c,
        out_type=jax.ShapeDtypeStruct((NC, n_pad, H), f32),
        mesh=mesh,
        scratch_types=[
            pltpu.VMEM((N,), jnp.int32),
            pltpu.VMEM((ECHUNK_C,), jnp.int32),
            pltpu.VMEM((ECHUNK_C,), jnp.int32),
            pltpu.VMEM((ECHUNK_C,), jnp.int32),
            pltpu.VMEM((ECHUNK_C, H), f32),
            pltpu.VMEM((ECHUNK_C, H), f32),
            pltpu.VMEM_SHARED((emb.shape[0], H), f32),
            pltpu.VMEM_SHARED((n_pad, H), f32),
            pltpu.SemaphoreType.DMA,
        ],
        compiler_params=pltpu.CompilerParams(needs_layout_passes=False),
    )
    agg = kern_c(w_mat, src_p, dst_p, z, nemb.astype(f32), zero_init)

    def body_d(ne_ref, a0_ref, a1_ref, w1_ref, w2_ref, b_ref, out_ref):
        acc = jnp.dot(ne_ref[...], w1_ref[...], preferred_element_type=f32)
        acc = acc + jnp.dot(a0_ref[...] + a1_ref[...], w2_ref[...],
                            preferred_element_type=f32)
        out_ref[...] = acc + b_ref[...]

    nbn = n_pad // 1024
    row_spec = pl.BlockSpec((1024, H), lambda i: (i, 0))
    node_emb = pl.pallas_call(
        body_d,
        grid=(nbn,),
        in_specs=[row_spec, row_spec, row_spec,
                  pl.BlockSpec((H, H), lambda i: (0, 0)),
                  pl.BlockSpec((H, H), lambda i: (0, 0)),
                  pl.BlockSpec((1, H), lambda i: (0, 0))],
        out_specs=row_spec,
        out_shape=jax.ShapeDtypeStruct((n_pad, H), f32),
    )(ne, agg[0], agg[1], comb_w.astype(f32)[:H], comb_w.astype(f32)[H:],
      comb_b.astype(f32)[None, :])

    node_embedding = node_emb[:N]
    node_vec = jnp.zeros((N, 3, H), f32)
    edge_weight = ew3.reshape(e_pad)[:E]
    edge_attr_out = attr_out[:E]
    edge_vec = jnp.stack([enx3.reshape(e_pad)[:E],
                          eny3.reshape(e_pad)[:E],
                          enz3.reshape(e_pad)[:E]], axis=-1)
    return (node_embedding, node_vec, edge_index, edge_weight,
            edge_attr_out, edge_vec)


# speedup vs baseline: 3.4099x; 1.1202x over previous
import math

import jax
import jax.numpy as jnp
from jax import lax
from jax.experimental import pallas as pl
from jax.experimental.pallas import tpu as pltpu
from jax.experimental.pallas import tpu_sc as plsc

CU = 5.0
CL = 0.0
NC = 2
NS = 16
LANES = 16
NW = NC * NS
RPAD = 64
ECHUNK_A = 1024
ECHUNK_C = 128
NCHUNK_A = 64


def _cdiv(a, b):
    return (a + b - 1) // b


def kernel(z, pos, edge_index, emb, means, betas, rbf_w, rbf_b, nemb,
           dist_w, dist_b, comb_w, comb_b):
    N = z.shape[0]
    E = edge_index.shape[1]
    H = emb.shape[1]
    R = means.shape[0]

    n_pad = _cdiv(N, NW * NCHUNK_A) * NW * NCHUNK_A
    e_pad = _cdiv(E, NW * ECHUNK_A) * NW * ECHUNK_A
    npw = n_pad // NW
    epw = e_pad // NW
    nb = e_pad // ECHUNK_A
    f32 = jnp.float32

    z = z.astype(jnp.int32)
    src = edge_index[0].astype(jnp.int32)
    dst = edge_index[1].astype(jnp.int32)
    z_p = jnp.pad(z, (0, n_pad - N))
    src_p = jnp.pad(src, (0, e_pad - E))
    dst_p = jnp.pad(dst, (0, e_pad - E))
    posf = jnp.pad(pos.astype(f32), ((0, 0), (0, 1))).reshape(-1)

    mesh = plsc.VectorSubcoreMesh(core_axis_name="c", subcore_axis_name="s")

    def body_a(z_hbm, posf_hbm, emb_hbm, src_hbm, dst_hbm,
               ne_hbm, evx_hbm, evy_hbm, evz_hbm, zsrc_hbm,
               zidx_v, nbuf_v, posf_v, zv, sidx_v, didx_v,
               ex_v, ey_v, ez_v, zs_v, sem):
        c = lax.axis_index("c")
        s = lax.axis_index("s")
        wid = s * NC + c
        base_n = wid * npw

        @pl.loop(0, npw // NCHUNK_A)
        def node_chunk(i):
            b = base_n + i * NCHUNK_A
            pltpu.sync_copy(z_hbm.at[pl.ds(b, NCHUNK_A)], zidx_v)
            pltpu.async_copy(emb_hbm.at[zidx_v], nbuf_v, sem).wait()
            pltpu.sync_copy(nbuf_v, ne_hbm.at[pl.ds(b, NCHUNK_A)])

        pltpu.sync_copy(posf_hbm, posf_v)
        pltpu.sync_copy(z_hbm, zv)
        base_e = wid * epw

        @pl.loop(0, epw // ECHUNK_A)
        def edge_chunk(k):
            b = base_e + k * ECHUNK_A
            pltpu.sync_copy(src_hbm.at[pl.ds(b, ECHUNK_A)], sidx_v)
            pltpu.sync_copy(dst_hbm.at[pl.ds(b, ECHUNK_A)], didx_v)

            @pl.loop(0, ECHUNK_A // LANES, unroll=4)
            def g(j):
                o = j * LANES
                s16 = sidx_v[pl.ds(o, LANES)]
                si = s16 * 4
                di = didx_v[pl.ds(o, LANES)] * 4
                ex_v[pl.ds(o, LANES)] = (plsc.load_gather(posf_v, [si]) -
                                         plsc.load_gather(posf_v, [di]))
                ey_v[pl.ds(o, LANES)] = (plsc.load_gather(posf_v, [si + 1]) -
                                         plsc.load_gather(posf_v, [di + 1]))
                ez_v[pl.ds(o, LANES)] = (plsc.load_gather(posf_v, [si + 2]) -
                                         plsc.load_gather(posf_v, [di + 2]))
                zs_v[pl.ds(o, LANES)] = plsc.load_gather(zv, [s16])

            pltpu.sync_copy(ex_v, evx_hbm.at[pl.ds(b, ECHUNK_A)])
            pltpu.sync_copy(ey_v, evy_hbm.at[pl.ds(b, ECHUNK_A)])
            pltpu.sync_copy(ez_v, evz_hbm.at[pl.ds(b, ECHUNK_A)])
            pltpu.sync_copy(zs_v, zsrc_hbm.at[pl.ds(b, ECHUNK_A)])

    kern_a = pl.kernel(
        body_a,
        out_type=[
            jax.ShapeDtypeStruct((n_pad, H), f32),
            jax.ShapeDtypeStruct((e_pad,), f32),
            jax.ShapeDtypeStruct((e_pad,), f32),
            jax.ShapeDtypeStruct((e_pad,), f32),
            jax.ShapeDtypeStruct((e_pad,), jnp.int32),
        ],
        mesh=mesh,
        scratch_types=[
            pltpu.VMEM((NCHUNK_A,), jnp.int32),
            pltpu.VMEM((NCHUNK_A, H), f32),
            pltpu.VMEM((4 * N,), f32),
            pltpu.VMEM((n_pad,), jnp.int32),
            pltpu.VMEM((ECHUNK_A,), jnp.int32),
            pltpu.VMEM((ECHUNK_A,), jnp.int32),
            pltpu.VMEM((ECHUNK_A,), f32),
            pltpu.VMEM((ECHUNK_A,), f32),
            pltpu.VMEM((ECHUNK_A,), f32),
            pltpu.VMEM((ECHUNK_A,), jnp.int32),
            pltpu.SemaphoreType.DMA,
        ],
        compiler_params=pltpu.CompilerParams(needs_layout_passes=False),
    )
    ne, evx, evy, evz, zsrc = kern_a(z_p, posf, emb.astype(f32), src_p, dst_p)

    meansb = jnp.broadcast_to(
        jnp.pad(means.astype(f32), (0, RPAD - R))[:, None], (RPAD, 128))
    betasb = jnp.broadcast_to(
        jnp.pad(betas.astype(f32), (0, RPAD - R))[:, None], (RPAD, 128))
    rbfa = jnp.concatenate(
        [rbf_w.astype(f32), jnp.zeros((RPAD - 1 - R, H), f32),
         rbf_b.astype(f32)[None, :]], axis=0)
    dista = jnp.concatenate(
        [dist_w.astype(f32), jnp.zeros((RPAD - 1 - R, H), f32),
         dist_b.astype(f32)[None, :]], axis=0)

    def body_b(ex_ref, ey_ref, ez_ref, srcb_ref, dstb_ref,
               means_ref, betas_ref, rbfw_ref, distw_ref,
               ew_ref, enx_ref, eny_ref, enz_ref, attr_ref, w_ref):
        ex = ex_ref[0]
        ey = ey_ref[0]
        ez = ez_ref[0]
        d2 = ex * ex + ey * ey + ez * ez
        d = jnp.sqrt(d2)
        ew_ref[0] = d
        inv = 1.0 / d
        enx_ref[0] = ex * inv
        eny_ref[0] = ey * inv
        enz_ref[0] = ez * inv
        cut = 0.5 * (jnp.cos(d * (math.pi / CU)) + 1.0) * (d < CU).astype(f32)
        neq = (srcb_ref[0] != dstb_ref[0]).astype(f32)
        cn = cut * neq
        mm = means_ref[...]
        bb = betas_ref[...]
        rowid = lax.broadcasted_iota(jnp.int32, (RPAD, 128), 0)
        alpha = 5.0 / (CU - CL)
        for j in range(8):
            dj = d[j:j + 1, :]
            attr = jnp.exp(-bb * (jnp.exp(alpha * (CL - dj)) - mm) ** 2)
            attr = attr * cut[j:j + 1, :]
            attr = jnp.where(rowid < R, attr, 0.0)
            attr = jnp.where(rowid == RPAD - 1, 1.0, attr)
            attr_ref[pl.ds(j * 128, 128), :] = lax.dot_general(
                attr, rbfw_ref[...], (((0,), (0,)), ((), ())),
                preferred_element_type=f32)
            attr2 = attr * cn[j:j + 1, :]
            w_ref[pl.ds(j * 128, 128), :] = lax.dot_general(
                attr2, distw_ref[...], (((0,), (0,)), ((), ())),
                preferred_element_type=f32)

    ev_spec = pl.BlockSpec((1, 8, 128), lambda i: (i, 0, 0))
    par_spec = pl.BlockSpec((RPAD, 128), lambda i: (0, 0))
    eh_spec = pl.BlockSpec((ECHUNK_A, H), lambda i: (i, 0))
    ew3, enx3, eny3, enz3, attr_out, w_mat = pl.pallas_call(
        body_b,
        grid=(nb,),
        in_specs=[ev_spec, ev_spec, ev_spec, ev_spec, ev_spec,
                  par_spec, par_spec, par_spec, par_spec],
        out_specs=[ev_spec, ev_spec, ev_spec, ev_spec, eh_spec, eh_spec],
        out_shape=[
            jax.ShapeDtypeStruct((nb, 8, 128), f32),
            jax.ShapeDtypeStruct((nb, 8, 128), f32),
            jax.ShapeDtypeStruct((nb, 8, 128), f32),
            jax.ShapeDtypeStruct((nb, 8, 128), f32),
            jax.ShapeDtypeStruct((e_pad, H), f32),
            jax.ShapeDtypeStruct((e_pad, H), f32),
        ],
    )(evx.reshape(nb, 8, 128), evy.reshape(nb, 8, 128),
      evz.reshape(nb, 8, 128), src_p.reshape(nb, 8, 128),
      dst_p.reshape(nb, 8, 128), meansb, betasb, rbfa, dista)

    zero_init = jnp.zeros((n_pad, H), f32)
    rpt = n_pad // NS
    CH = 32
    nch = epw // CH
    zsrc2d = zsrc.reshape(e_pad // CH, CH)
    dst2d = dst_p.reshape(e_pad // CH, CH)

    def body_c(w_hbm, zsrc_hbm, dst_hbm, nemb_hbm, zero_hbm,
               agg_hbm,
               xb0, xb1, xb2, xb3, wb0, wb1, wb2, wb3,
               zi0, zi1, zi2, zi3, db0, db1, db2, db3,
               nemb_sh, agg_sh,
               g0, g1, g2, g3, w0, w1, w2, w3,
               z0, z1, z2, z3, d0, d1, d2, d3,
               s0, s1, s2, s3):
        c = lax.axis_index("c")
        s = lax.axis_index("s")
        wid = s * NC + c
        pltpu.sync_copy(zero_hbm.at[pl.ds(s * rpt, rpt)],
                        agg_sh.at[pl.ds(s * rpt, rpt)])

        @pl.when(s == 0)
        def _():
            pltpu.sync_copy(nemb_hbm, nemb_sh)

        plsc.subcore_barrier()
        base_r = wid * nch
        base_e = wid * epw
        xb = (xb0, xb1, xb2, xb3)
        wb = (wb0, wb1, wb2, wb3)
        zi = (zi0, zi1, zi2, zi3)
        db = (db0, db1, db2, db3)
        gsem = (g0, g1, g2, g3)
        wsem = (w0, w1, w2, w3)
        zsem = (z0, z1, z2, z3)
        dsem = (d0, d1, d2, d3)
        ssem = (s0, s1, s2, s3)

        def fire_inputs(g, b):
            pltpu.async_copy(w_hbm.at[pl.ds(base_e + g * CH, CH)],
                             wb[b], wsem[b])
            pltpu.async_copy(zsrc_hbm.at[base_r + g], zi[b], zsem[b])
            pltpu.async_copy(dst_hbm.at[base_r + g], db[b], dsem[b])

        def wait_inputs_idx(g, b):
            pltpu.make_async_copy(zsrc_hbm.at[base_r + g], zi[b],
                                  zsem[b]).wait()

        for b in range(2):
            fire_inputs(b, b)
        wait_inputs_idx(0, 0)
        pltpu.async_copy(nemb_sh.at[zi[0]], xb[0], gsem[0])

        @pl.loop(0, nch // 4)
        def quad(p):
            for b in range(4):
                g = p * 4 + b
                b1 = (b + 1) % 4
                b2 = (b + 2) % 4

                @pl.when(g + 1 < nch)
                def _():
                    wait_inputs_idx(g + 1, b1)
                    pltpu.async_copy(nemb_sh.at[zi[b1]], xb[b1], gsem[b1])

                pltpu.make_async_copy(nemb_sh.at[zi[b]], xb[b],
                                      gsem[b]).wait()
                pltpu.make_async_copy(
                    w_hbm.at[pl.ds(base_e + g * CH, CH)], wb[b],
                    wsem[b]).wait()
                pltpu.make_async_copy(dst_hbm.at[base_r + g], db[b],
                                      dsem[b]).wait()

                @pl.loop(0, CH, unroll=4)
                def mul(r):
                    for h in range(H // LANES):
                        o = h * LANES
                        xb[b][r, pl.ds(o, LANES)] = (
                            xb[b][r, pl.ds(o, LANES)] *
                            wb[b][r, pl.ds(o, LANES)])

                pltpu.async_copy(xb[b], agg_sh.at[db[b]], ssem[b], add=True)

                @pl.when(g + 2 < nch)
                def _():
                    @pl.when(g >= 2)
                    def _():
                        pltpu.make_async_copy(xb[b2], agg_sh.at[db[b2]],
                                              ssem[b2]).wait()
                    fire_inputs(g + 2, b2)

        for gl in (nch - 4, nch - 3, nch - 2, nch - 1):
            b = gl % 4
            pltpu.make_async_copy(xb[b], agg_sh.at[db[b]], ssem[b]).wait()

        plsc.subcore_barrier()
        pltpu.sync_copy(agg_sh.at[pl.ds(s * rpt, rpt)],
                        agg_hbm.at[c, pl.ds(s * rpt, rpt)])

    dma = pltpu.SemaphoreType.DMA
    kern_c = pl.kernel(
        body_c,
        out_type=jax.ShapeDtypeStruct((NC, n_pad, H), f32),
        mesh=mesh,
        scratch_types=(
            [pltpu.VMEM((CH, H), f32)] * 8 +
            [pltpu.VMEM((CH,), jnp.int32)] * 8 +
            [pltpu.VMEM_SHARED((emb.shape[0], H), f32),
             pltpu.VMEM_SHARED((n_pad, H), f32)] +
            [dma] * 20),
        compiler_params=pltpu.CompilerParams(needs_layout_passes=False),
    )
    agg = kern_c(w_mat, zsrc2d, dst2d, nemb.astype(f32), zero_init)

    def body_d(ne_ref, a0_ref, a1_ref, w1_ref, w2_ref, b_ref, out_ref):
        acc = jnp.dot(ne_ref[...], w1_ref[...], preferred_element_type=f32)
        acc = acc + jnp.dot(a0_ref[...] + a1_ref[...], w2_ref[...],
                            preferred_element_type=f32)
        out_ref[...] = acc + b_ref[...]

    nbn = n_pad // 1024
    row_spec = pl.BlockSpec((1024, H), lambda i: (i, 0))
    node_emb = pl.pallas_call(
        body_d,
        grid=(nbn,),
        in_specs=[row_spec, row_spec, row_spec,
                  pl.BlockSpec((H, H), lambda i: (0, 0)),
                  pl.BlockSpec((H, H), lambda i: (0, 0)),
                  pl.BlockSpec((1, H), lambda i: (0, 0))],
        out_specs=row_spec,
        out_shape=jax.ShapeDtypeStruct((n_pad, H), f32),
    )(ne, agg[0], agg[1], comb_w.astype(f32)[:H], comb_w.astype(f32)[H:],
      comb_b.astype(f32)[None, :])

    node_embedding = node_emb[:N]
    node_vec = jnp.zeros((N, 3, H), f32)
    edge_weight = ew3.reshape(e_pad)[:E]
    edge_attr_out = attr_out[:E]
    edge_vec = jnp.stack([enx3.reshape(e_pad)[:E],
                          eny3.reshape(e_pad)[:E],
                          enz3.reshape(e_pad)[:E]], axis=-1)
    return (node_embedding, node_vec, edge_index, edge_weight,
            edge_attr_out, edge_vec)


# speedup vs baseline: 6.3466x; 1.8613x over previous
import math

import jax
import jax.numpy as jnp
from jax import lax
from jax.experimental import pallas as pl
from jax.experimental.pallas import tpu as pltpu
from jax.experimental.pallas import tpu_sc as plsc

CU = 5.0
CL = 0.0
NC = 2
NS = 16
LANES = 16
NW = NC * NS
RPAD = 64
ECHUNK_A = 1024
ECHUNK_C = 128
NCHUNK_A = 64


def _cdiv(a, b):
    return (a + b - 1) // b


def kernel(z, pos, edge_index, emb, means, betas, rbf_w, rbf_b, nemb,
           dist_w, dist_b, comb_w, comb_b):
    N = z.shape[0]
    E = edge_index.shape[1]
    H = emb.shape[1]
    R = means.shape[0]

    n_pad = _cdiv(N, NW * NCHUNK_A) * NW * NCHUNK_A
    e_pad = _cdiv(E, NW * ECHUNK_A) * NW * ECHUNK_A
    npw = n_pad // NW
    epw = e_pad // NW
    nb = e_pad // ECHUNK_A
    f32 = jnp.float32

    z = z.astype(jnp.int32)
    src = edge_index[0].astype(jnp.int32)
    dst = edge_index[1].astype(jnp.int32)
    z_p = jnp.pad(z, (0, n_pad - N))
    src_p = jnp.pad(src, (0, e_pad - E))
    dst_p = jnp.pad(dst, (0, e_pad - E))
    posf = jnp.pad(pos.astype(f32), ((0, 0), (0, 1))).reshape(-1)

    mesh = plsc.VectorSubcoreMesh(core_axis_name="c", subcore_axis_name="s")

    def body_a(z_hbm, posf_hbm, emb_hbm, src_hbm, dst_hbm,
               ne_hbm, evx_hbm, evy_hbm, evz_hbm, zsrc_hbm,
               zidx_v, nbuf_v, posf_v, zv, sidx_v, didx_v,
               ex_v, ey_v, ez_v, zs_v, sem):
        c = lax.axis_index("c")
        s = lax.axis_index("s")
        wid = s * NC + c
        base_n = wid * npw

        @pl.loop(0, npw // NCHUNK_A)
        def node_chunk(i):
            b = base_n + i * NCHUNK_A
            pltpu.sync_copy(z_hbm.at[pl.ds(b, NCHUNK_A)], zidx_v)
            pltpu.async_copy(emb_hbm.at[zidx_v], nbuf_v, sem).wait()
            pltpu.sync_copy(nbuf_v, ne_hbm.at[pl.ds(b, NCHUNK_A)])

        pltpu.sync_copy(posf_hbm, posf_v)
        pltpu.sync_copy(z_hbm, zv)
        base_e = wid * epw

        @pl.loop(0, epw // ECHUNK_A)
        def edge_chunk(k):
            b = base_e + k * ECHUNK_A
            pltpu.sync_copy(src_hbm.at[pl.ds(b, ECHUNK_A)], sidx_v)
            pltpu.sync_copy(dst_hbm.at[pl.ds(b, ECHUNK_A)], didx_v)

            @pl.loop(0, ECHUNK_A // LANES, unroll=4)
            def g(j):
                o = j * LANES
                s16 = sidx_v[pl.ds(o, LANES)]
                si = s16 * 4
                di = didx_v[pl.ds(o, LANES)] * 4
                ex_v[pl.ds(o, LANES)] = (plsc.load_gather(posf_v, [si]) -
                                         plsc.load_gather(posf_v, [di]))
                ey_v[pl.ds(o, LANES)] = (plsc.load_gather(posf_v, [si + 1]) -
                                         plsc.load_gather(posf_v, [di + 1]))
                ez_v[pl.ds(o, LANES)] = (plsc.load_gather(posf_v, [si + 2]) -
                                         plsc.load_gather(posf_v, [di + 2]))
                zs_v[pl.ds(o, LANES)] = plsc.load_gather(zv, [s16])

            pltpu.sync_copy(ex_v, evx_hbm.at[pl.ds(b, ECHUNK_A)])
            pltpu.sync_copy(ey_v, evy_hbm.at[pl.ds(b, ECHUNK_A)])
            pltpu.sync_copy(ez_v, evz_hbm.at[pl.ds(b, ECHUNK_A)])
            pltpu.sync_copy(zs_v, zsrc_hbm.at[pl.ds(b, ECHUNK_A)])

    kern_a = pl.kernel(
        body_a,
        out_type=[
            jax.ShapeDtypeStruct((n_pad, H), f32),
            jax.ShapeDtypeStruct((e_pad,), f32),
            jax.ShapeDtypeStruct((e_pad,), f32),
            jax.ShapeDtypeStruct((e_pad,), f32),
            jax.ShapeDtypeStruct((e_pad,), jnp.int32),
        ],
        mesh=mesh,
        scratch_types=[
            pltpu.VMEM((NCHUNK_A,), jnp.int32),
            pltpu.VMEM((NCHUNK_A, H), f32),
            pltpu.VMEM((4 * N,), f32),
            pltpu.VMEM((n_pad,), jnp.int32),
            pltpu.VMEM((ECHUNK_A,), jnp.int32),
            pltpu.VMEM((ECHUNK_A,), jnp.int32),
            pltpu.VMEM((ECHUNK_A,), f32),
            pltpu.VMEM((ECHUNK_A,), f32),
            pltpu.VMEM((ECHUNK_A,), f32),
            pltpu.VMEM((ECHUNK_A,), jnp.int32),
            pltpu.SemaphoreType.DMA,
        ],
        compiler_params=pltpu.CompilerParams(needs_layout_passes=False),
    )
    ne, evx, evy, evz, zsrc = kern_a(z_p, posf, emb.astype(f32), src_p, dst_p)

    meansb = jnp.broadcast_to(
        jnp.pad(means.astype(f32), (0, RPAD - R))[:, None], (RPAD, 128))
    betasb = jnp.broadcast_to(
        jnp.pad(betas.astype(f32), (0, RPAD - R))[:, None], (RPAD, 128))
    rbfa = jnp.concatenate(
        [rbf_w.astype(f32), jnp.zeros((RPAD - 1 - R, H), f32),
         rbf_b.astype(f32)[None, :]], axis=0)
    dista = jnp.concatenate(
        [dist_w.astype(f32), jnp.zeros((RPAD - 1 - R, H), f32),
         dist_b.astype(f32)[None, :]], axis=0)

    maxz = emb.shape[0]
    zpad = _cdiv(max(maxz, 128), 128) * 128
    nembp = jnp.zeros((zpad, H), f32).at[:maxz].set(nemb.astype(f32))

    def body_b(ex_ref, ey_ref, ez_ref, srcb_ref, dstb_ref, zsrcb_ref,
               means_ref, betas_ref, rbfw_ref, distw_ref, nemb_ref,
               ew_ref, enx_ref, eny_ref, enz_ref, attr_ref, msg_ref):
        ex = ex_ref[0]
        ey = ey_ref[0]
        ez = ez_ref[0]
        d2 = ex * ex + ey * ey + ez * ez
        d = jnp.sqrt(d2)
        ew_ref[0] = d
        inv = 1.0 / d
        enx_ref[0] = ex * inv
        eny_ref[0] = ey * inv
        enz_ref[0] = ez * inv
        cut = 0.5 * (jnp.cos(d * (math.pi / CU)) + 1.0) * (d < CU).astype(f32)
        neq = (srcb_ref[0] != dstb_ref[0]).astype(f32)
        cn = cut * neq
        mm = means_ref[...]
        bb = betas_ref[...]
        rowid = lax.broadcasted_iota(jnp.int32, (RPAD, 128), 0)
        zrow = lax.broadcasted_iota(jnp.int32, (zpad, 128), 0)
        alpha = 5.0 / (CU - CL)
        for j in range(8):
            dj = d[j:j + 1, :]
            attr = jnp.exp(-bb * (jnp.exp(alpha * (CL - dj)) - mm) ** 2)
            attr = attr * cut[j:j + 1, :]
            attr = jnp.where(rowid < R, attr, 0.0)
            attr = jnp.where(rowid == RPAD - 1, 1.0, attr)
            attr_ref[pl.ds(j * 128, 128), :] = lax.dot_general(
                attr, rbfw_ref[...], (((0,), (0,)), ((), ())),
                preferred_element_type=f32)
            attr2 = attr * cn[j:j + 1, :]
            w_tile = lax.dot_general(
                attr2, distw_ref[...], (((0,), (0,)), ((), ())),
                preferred_element_type=f32)
            oh = (zrow == zsrcb_ref[0][j:j + 1, :]).astype(f32)
            xsrc = lax.dot_general(
                oh, nemb_ref[...], (((0,), (0,)), ((), ())),
                preferred_element_type=f32)
            msg_ref[pl.ds(j * 128, 128), :] = xsrc * w_tile

    nb2 = _cdiv(E, ECHUNK_A)
    e2 = nb2 * ECHUNK_A
    ev_spec = pl.BlockSpec((1, 8, 128), lambda i: (i, 0, 0))
    par_spec = pl.BlockSpec((RPAD, 128), lambda i: (0, 0))
    eh_spec = pl.BlockSpec((ECHUNK_A, H), lambda i: (i, 0))
    ew3, enx3, eny3, enz3, attr_out, msg = pl.pallas_call(
        body_b,
        grid=(nb2,),
        in_specs=[ev_spec, ev_spec, ev_spec, ev_spec, ev_spec, ev_spec,
                  par_spec, par_spec, par_spec, par_spec,
                  pl.BlockSpec((zpad, 128), lambda i: (0, 0))],
        out_specs=[ev_spec, ev_spec, ev_spec, ev_spec, eh_spec, eh_spec],
        out_shape=[
            jax.ShapeDtypeStruct((nb2, 8, 128), f32),
            jax.ShapeDtypeStruct((nb2, 8, 128), f32),
            jax.ShapeDtypeStruct((nb2, 8, 128), f32),
            jax.ShapeDtypeStruct((nb2, 8, 128), f32),
            jax.ShapeDtypeStruct((E, H), f32),
            jax.ShapeDtypeStruct((E, H), f32),
        ],
    )(evx[:e2].reshape(nb2, 8, 128), evy[:e2].reshape(nb2, 8, 128),
      evz[:e2].reshape(nb2, 8, 128), src_p[:e2].reshape(nb2, 8, 128),
      dst_p[:e2].reshape(nb2, 8, 128), zsrc[:e2].reshape(nb2, 8, 128),
      meansb, betasb, rbfa, dista, nembp)

    zero_init = jnp.zeros((n_pad, H), f32)
    rpt = n_pad // NS
    CH = 80
    epc = E // NW
    nch = epc // CH
    dst2d = dst.reshape(E // CH, CH)

    def body_c(msg_hbm, dst_hbm, zero_hbm,
               agg_hbm,
               mb0, mb1, mb2, mb3, db0, db1, db2, db3,
               agg_sh,
               m0, m1, m2, m3, d0, d1, d2, d3,
               s0, s1, s2, s3):
        c = lax.axis_index("c")
        s = lax.axis_index("s")
        wid = s * NC + c
        pltpu.sync_copy(zero_hbm.at[pl.ds(s * rpt, rpt)],
                        agg_sh.at[pl.ds(s * rpt, rpt)])
        plsc.subcore_barrier()
        base_r = wid * nch
        base_e = wid * epc
        mb = (mb0, mb1, mb2, mb3)
        db = (db0, db1, db2, db3)
        msem = (m0, m1, m2, m3)
        dsem = (d0, d1, d2, d3)
        ssem = (s0, s1, s2, s3)

        def fire_inputs(g, b):
            pltpu.async_copy(msg_hbm.at[pl.ds(base_e + g * CH, CH)],
                             mb[b], msem[b])
            pltpu.async_copy(dst_hbm.at[base_r + g], db[b], dsem[b])

        for b in range(2):
            fire_inputs(b, b)

        def run_chunk(g, b, refill):
            b2 = (b + 2) % 4
            pltpu.make_async_copy(
                msg_hbm.at[pl.ds(base_e + g * CH, CH)], mb[b],
                msem[b]).wait()
            pltpu.make_async_copy(dst_hbm.at[base_r + g], db[b],
                                  dsem[b]).wait()
            pltpu.async_copy(mb[b], agg_sh.at[db[b]], ssem[b], add=True)

            if refill:
                @pl.when(g + 2 < nch)
                def _():
                    @pl.when(g >= 2)
                    def _():
                        pltpu.make_async_copy(mb[b2], agg_sh.at[db[b2]],
                                              ssem[b2]).wait()
                    fire_inputs(g + 2, b2)

        @pl.loop(0, nch // 4)
        def quad(p):
            for b in range(4):
                run_chunk(p * 4 + b, b, True)

        for gr in range(nch - (nch % 4), nch):
            run_chunk(gr, gr % 4, False)

        for gl in (nch - 4, nch - 3, nch - 2, nch - 1):
            b = gl % 4
            pltpu.make_async_copy(mb[b], agg_sh.at[db[b]], ssem[b]).wait()

        plsc.subcore_barrier()
        pltpu.sync_copy(agg_sh.at[pl.ds(s * rpt, rpt)],
                        agg_hbm.at[c, pl.ds(s * rpt, rpt)])

    dma = pltpu.SemaphoreType.DMA
    kern_c = pl.kernel(
        body_c,
        out_type=jax.ShapeDtypeStruct((NC, n_pad, H), f32),
        mesh=mesh,
        scratch_types=(
            [pltpu.VMEM((CH, H), f32)] * 4 +
            [pltpu.VMEM((CH,), jnp.int32)] * 4 +
            [pltpu.VMEM_SHARED((n_pad, H), f32)] +
            [dma] * 12),
        compiler_params=pltpu.CompilerParams(needs_layout_passes=False),
    )
    agg = kern_c(msg, dst2d, zero_init)

    def body_d(ne_ref, a0_ref, a1_ref, w1_ref, w2_ref, b_ref, out_ref):
        acc = jnp.dot(ne_ref[...], w1_ref[...], preferred_element_type=f32)
        acc = acc + jnp.dot(a0_ref[...] + a1_ref[...], w2_ref[...],
                            preferred_element_type=f32)
        out_ref[...] = acc + b_ref[...]

    nbn = n_pad // 1024
    row_spec = pl.BlockSpec((1024, H), lambda i: (i, 0))
    node_emb = pl.pallas_call(
        body_d,
        grid=(nbn,),
        in_specs=[row_spec, row_spec, row_spec,
                  pl.BlockSpec((H, H), lambda i: (0, 0)),
                  pl.BlockSpec((H, H), lambda i: (0, 0)),
                  pl.BlockSpec((1, H), lambda i: (0, 0))],
        out_specs=row_spec,
        out_shape=jax.ShapeDtypeStruct((n_pad, H), f32),
    )(ne, agg[0], agg[1], comb_w.astype(f32)[:H], comb_w.astype(f32)[H:],
      comb_b.astype(f32)[None, :])

    node_embedding = node_emb[:N]
    node_vec = jnp.zeros((N, 3, H), f32)
    edge_weight = ew3.reshape(e2)[:E]
    edge_attr_out = attr_out
    edge_vec = jnp.stack([enx3.reshape(e2)[:E],
                          eny3.reshape(e2)[:E],
                          enz3.reshape(e2)[:E]], axis=-1)
    return (node_embedding, node_vec, edge_index, edge_weight,
            edge_attr_out, edge_vec)


# speedup vs baseline: 6.4968x; 1.0237x over previous
import math

import jax
import jax.numpy as jnp
from jax import lax
from jax.experimental import pallas as pl
from jax.experimental.pallas import tpu as pltpu
from jax.experimental.pallas import tpu_sc as plsc

CU = 5.0
CL = 0.0
NC = 2
NS = 16
LANES = 16
NW = NC * NS
RPAD = 64
ECHUNK_A = 1024
ECHUNK_C = 128
NCHUNK_A = 64


def _cdiv(a, b):
    return (a + b - 1) // b


def kernel(z, pos, edge_index, emb, means, betas, rbf_w, rbf_b, nemb,
           dist_w, dist_b, comb_w, comb_b):
    N = z.shape[0]
    E = edge_index.shape[1]
    H = emb.shape[1]
    R = means.shape[0]

    n_pad = _cdiv(N, NW * NCHUNK_A) * NW * NCHUNK_A
    e_pad = _cdiv(E, NW * ECHUNK_A) * NW * ECHUNK_A
    npw = n_pad // NW
    epw = e_pad // NW
    nb = e_pad // ECHUNK_A
    f32 = jnp.float32

    z = z.astype(jnp.int32)
    src = edge_index[0].astype(jnp.int32)
    dst = edge_index[1].astype(jnp.int32)
    z_p = jnp.pad(z, (0, n_pad - N))
    src_p = jnp.pad(src, (0, e_pad - E))
    dst_p = jnp.pad(dst, (0, e_pad - E))
    posf = jnp.pad(pos.astype(f32), ((0, 0), (0, 1))).reshape(-1)

    mesh = plsc.VectorSubcoreMesh(core_axis_name="c", subcore_axis_name="s")

    def body_a(z_hbm, posf_hbm, emb_hbm, src_hbm, dst_hbm,
               ne_hbm, evx_hbm, evy_hbm, evz_hbm, zsrc_hbm,
               zidx_v, nbuf_v, posf_v, zv,
               si0, si1, di0, di1,
               exb0, exb1, eyb0, eyb1, ezb0, ezb1, zsb0, zsb1,
               sem, is0, is1, os0, os1):
        c = lax.axis_index("c")
        s = lax.axis_index("s")
        wid = s * NC + c
        base_n = wid * npw

        @pl.loop(0, npw // NCHUNK_A)
        def node_chunk(i):
            b = base_n + i * NCHUNK_A
            pltpu.sync_copy(z_hbm.at[pl.ds(b, NCHUNK_A)], zidx_v)
            pltpu.async_copy(emb_hbm.at[zidx_v], nbuf_v, sem).wait()
            pltpu.sync_copy(nbuf_v, ne_hbm.at[pl.ds(b, NCHUNK_A)])

        pltpu.sync_copy(posf_hbm, posf_v)
        pltpu.sync_copy(z_hbm, zv)
        base_e = wid * epw
        nch_a = epw // ECHUNK_A
        si = (si0, si1)
        di = (di0, di1)
        exb = (exb0, exb1)
        eyb = (eyb0, eyb1)
        ezb = (ezb0, ezb1)
        zsb = (zsb0, zsb1)
        isem = (is0, is1)
        osem = (os0, os1)

        def fire_idx(k, b):
            e0 = base_e + k * ECHUNK_A
            pltpu.async_copy(src_hbm.at[pl.ds(e0, ECHUNK_A)], si[b], isem[b])
            pltpu.async_copy(dst_hbm.at[pl.ds(e0, ECHUNK_A)], di[b], isem[b])

        def out_descs(k, b):
            e0 = base_e + k * ECHUNK_A
            return [
                pltpu.make_async_copy(exb[b], evx_hbm.at[pl.ds(e0, ECHUNK_A)],
                                      osem[b]),
                pltpu.make_async_copy(eyb[b], evy_hbm.at[pl.ds(e0, ECHUNK_A)],
                                      osem[b]),
                pltpu.make_async_copy(ezb[b], evz_hbm.at[pl.ds(e0, ECHUNK_A)],
                                      osem[b]),
                pltpu.make_async_copy(zsb[b], zsrc_hbm.at[pl.ds(e0, ECHUNK_A)],
                                      osem[b]),
            ]

        for b in range(2):
            fire_idx(b, b)

        @pl.loop(0, nch_a // 2)
        def edge_pair(p):
            for b in range(2):
                k = p * 2 + b
                e0 = base_e + k * ECHUNK_A
                pltpu.make_async_copy(src_hbm.at[pl.ds(e0, ECHUNK_A)],
                                      si[b], isem[b]).wait()
                pltpu.make_async_copy(dst_hbm.at[pl.ds(e0, ECHUNK_A)],
                                      di[b], isem[b]).wait()

                @pl.when(k >= 2)
                def _():
                    for dsc in out_descs(k - 2, b):
                        dsc.wait()

                @pl.loop(0, ECHUNK_A // LANES, unroll=4)
                def g(j):
                    o = j * LANES
                    s16 = si[b][pl.ds(o, LANES)]
                    sx = s16 * 4
                    dx = di[b][pl.ds(o, LANES)] * 4
                    exb[b][pl.ds(o, LANES)] = (
                        plsc.load_gather(posf_v, [sx]) -
                        plsc.load_gather(posf_v, [dx]))
                    eyb[b][pl.ds(o, LANES)] = (
                        plsc.load_gather(posf_v, [sx + 1]) -
                        plsc.load_gather(posf_v, [dx + 1]))
                    ezb[b][pl.ds(o, LANES)] = (
                        plsc.load_gather(posf_v, [sx + 2]) -
                        plsc.load_gather(posf_v, [dx + 2]))
                    zsb[b][pl.ds(o, LANES)] = plsc.load_gather(zv, [s16])

                pltpu.async_copy(exb[b], evx_hbm.at[pl.ds(e0, ECHUNK_A)],
                                 osem[b])
                pltpu.async_copy(eyb[b], evy_hbm.at[pl.ds(e0, ECHUNK_A)],
                                 osem[b])
                pltpu.async_copy(ezb[b], evz_hbm.at[pl.ds(e0, ECHUNK_A)],
                                 osem[b])
                pltpu.async_copy(zsb[b], zsrc_hbm.at[pl.ds(e0, ECHUNK_A)],
                                 osem[b])

                @pl.when(k + 2 < nch_a)
                def _():
                    fire_idx(k + 2, b)

        for kl in (nch_a - 2, nch_a - 1):
            for dsc in out_descs(kl, kl % 2):
                dsc.wait()

    kern_a = pl.kernel(
        body_a,
        out_type=[
            jax.ShapeDtypeStruct((n_pad, H), f32),
            jax.ShapeDtypeStruct((e_pad,), f32),
            jax.ShapeDtypeStruct((e_pad,), f32),
            jax.ShapeDtypeStruct((e_pad,), f32),
            jax.ShapeDtypeStruct((e_pad,), jnp.int32),
        ],
        mesh=mesh,
        scratch_types=[
            pltpu.VMEM((NCHUNK_A,), jnp.int32),
            pltpu.VMEM((NCHUNK_A, H), f32),
            pltpu.VMEM((4 * N,), f32),
            pltpu.VMEM((n_pad,), jnp.int32),
            pltpu.VMEM((ECHUNK_A,), jnp.int32),
            pltpu.VMEM((ECHUNK_A,), jnp.int32),
            pltpu.VMEM((ECHUNK_A,), jnp.int32),
            pltpu.VMEM((ECHUNK_A,), jnp.int32),
            pltpu.VMEM((ECHUNK_A,), f32),
            pltpu.VMEM((ECHUNK_A,), f32),
            pltpu.VMEM((ECHUNK_A,), f32),
            pltpu.VMEM((ECHUNK_A,), f32),
            pltpu.VMEM((ECHUNK_A,), f32),
            pltpu.VMEM((ECHUNK_A,), f32),
            pltpu.VMEM((ECHUNK_A,), jnp.int32),
            pltpu.VMEM((ECHUNK_A,), jnp.int32),
            pltpu.SemaphoreType.DMA,
            pltpu.SemaphoreType.DMA,
            pltpu.SemaphoreType.DMA,
            pltpu.SemaphoreType.DMA,
            pltpu.SemaphoreType.DMA,
        ],
        compiler_params=pltpu.CompilerParams(needs_layout_passes=False),
    )
    ne, evx, evy, evz, zsrc = kern_a(z_p, posf, emb.astype(f32), src_p, dst_p)

    meansb = jnp.broadcast_to(
        jnp.pad(means.astype(f32), (0, RPAD - R))[:, None], (RPAD, 128))
    betasb = jnp.broadcast_to(
        jnp.pad(betas.astype(f32), (0, RPAD - R))[:, None], (RPAD, 128))
    rbfa = jnp.concatenate(
        [rbf_w.astype(f32), jnp.zeros((RPAD - 1 - R, H), f32),
         rbf_b.astype(f32)[None, :]], axis=0)
    dista = jnp.concatenate(
        [dist_w.astype(f32), jnp.zeros((RPAD - 1 - R, H), f32),
         dist_b.astype(f32)[None, :]], axis=0)

    maxz = emb.shape[0]
    zpad = _cdiv(max(maxz, 128), 128) * 128
    nembp = jnp.zeros((zpad, H), f32).at[:maxz].set(nemb.astype(f32))

    def body_b(ex_ref, ey_ref, ez_ref, srcb_ref, dstb_ref, zsrcb_ref,
               means_ref, betas_ref, rbfw_ref, distw_ref, nemb_ref,
               ew_ref, enx_ref, eny_ref, enz_ref, attr_ref, msg_ref):
        ex = ex_ref[0]
        ey = ey_ref[0]
        ez = ez_ref[0]
        d2 = ex * ex + ey * ey + ez * ez
        d = jnp.sqrt(d2)
        ew_ref[0] = d
        inv = 1.0 / d
        enx_ref[0] = ex * inv
        eny_ref[0] = ey * inv
        enz_ref[0] = ez * inv
        cut = 0.5 * (jnp.cos(d * (math.pi / CU)) + 1.0) * (d < CU).astype(f32)
        neq = (srcb_ref[0] != dstb_ref[0]).astype(f32)
        cn = cut * neq
        mm = means_ref[...]
        bb = betas_ref[...]
        rowid = lax.broadcasted_iota(jnp.int32, (RPAD, 128), 0)
        zrow = lax.broadcasted_iota(jnp.int32, (zpad, 128), 0)
        alpha = 5.0 / (CU - CL)
        for j in range(8):
            dj = d[j:j + 1, :]
            attr = jnp.exp(-bb * (jnp.exp(alpha * (CL - dj)) - mm) ** 2)
            attr = attr * cut[j:j + 1, :]
            attr = jnp.where(rowid < R, attr, 0.0)
            attr = jnp.where(rowid == RPAD - 1, 1.0, attr)
            attr_ref[pl.ds(j * 128, 128), :] = lax.dot_general(
                attr, rbfw_ref[...], (((0,), (0,)), ((), ())),
                preferred_element_type=f32)
            attr2 = attr * cn[j:j + 1, :]
            w_tile = lax.dot_general(
                attr2, distw_ref[...], (((0,), (0,)), ((), ())),
                preferred_element_type=f32)
            oh = (zrow == zsrcb_ref[0][j:j + 1, :]).astype(f32)
            xsrc = lax.dot_general(
                oh, nemb_ref[...], (((0,), (0,)), ((), ())),
                preferred_element_type=f32)
            msg_ref[pl.ds(j * 128, 128), :] = xsrc * w_tile

    nb2 = _cdiv(E, ECHUNK_A)
    e2 = nb2 * ECHUNK_A
    ev_spec = pl.BlockSpec((1, 8, 128), lambda i: (i, 0, 0))
    par_spec = pl.BlockSpec((RPAD, 128), lambda i: (0, 0))
    eh_spec = pl.BlockSpec((ECHUNK_A, H), lambda i: (i, 0))
    ew3, enx3, eny3, enz3, attr_out, msg = pl.pallas_call(
        body_b,
        grid=(nb2,),
        in_specs=[ev_spec, ev_spec, ev_spec, ev_spec, ev_spec, ev_spec,
                  par_spec, par_spec, par_spec, par_spec,
                  pl.BlockSpec((zpad, 128), lambda i: (0, 0))],
        out_specs=[ev_spec, ev_spec, ev_spec, ev_spec, eh_spec, eh_spec],
        out_shape=[
            jax.ShapeDtypeStruct((nb2, 8, 128), f32),
            jax.ShapeDtypeStruct((nb2, 8, 128), f32),
            jax.ShapeDtypeStruct((nb2, 8, 128), f32),
            jax.ShapeDtypeStruct((nb2, 8, 128), f32),
            jax.ShapeDtypeStruct((E, H), f32),
            jax.ShapeDtypeStruct((E, H), f32),
        ],
    )(evx[:e2].reshape(nb2, 8, 128), evy[:e2].reshape(nb2, 8, 128),
      evz[:e2].reshape(nb2, 8, 128), src_p[:e2].reshape(nb2, 8, 128),
      dst_p[:e2].reshape(nb2, 8, 128), zsrc[:e2].reshape(nb2, 8, 128),
      meansb, betasb, rbfa, dista, nembp)

    zero_init = jnp.zeros((n_pad, H), f32)
    rpt = n_pad // NS
    CH = 80
    epc = E // NW
    nch = epc // CH
    dst2d = dst.reshape(E // CH, CH)

    def body_c(msg_hbm, dst_hbm, zero_hbm,
               agg_hbm,
               mb0, mb1, mb2, mb3, db0, db1, db2, db3,
               agg_sh,
               m0, m1, m2, m3, d0, d1, d2, d3,
               s0, s1, s2, s3):
        c = lax.axis_index("c")
        s = lax.axis_index("s")
        wid = s * NC + c
        pltpu.sync_copy(zero_hbm.at[pl.ds(s * rpt, rpt)],
                        agg_sh.at[pl.ds(s * rpt, rpt)])
        plsc.subcore_barrier()
        base_r = wid * nch
        base_e = wid * epc
        mb = (mb0, mb1, mb2, mb3)
        db = (db0, db1, db2, db3)
        msem = (m0, m1, m2, m3)
        dsem = (d0, d1, d2, d3)
        ssem = (s0, s1, s2, s3)

        def fire_inputs(g, b):
            pltpu.async_copy(msg_hbm.at[pl.ds(base_e + g * CH, CH)],
                             mb[b], msem[b])
            pltpu.async_copy(dst_hbm.at[base_r + g], db[b], dsem[b])

        for b in range(2):
            fire_inputs(b, b)

        def run_chunk(g, b, refill):
            b2 = (b + 2) % 4
            pltpu.make_async_copy(
                msg_hbm.at[pl.ds(base_e + g * CH, CH)], mb[b],
                msem[b]).wait()
            pltpu.make_async_copy(dst_hbm.at[base_r + g], db[b],
                                  dsem[b]).wait()
            pltpu.async_copy(mb[b], agg_sh.at[db[b]], ssem[b], add=True)

            if refill:
                @pl.when(g + 2 < nch)
                def _():
                    @pl.when(g >= 2)
                    def _():
                        pltpu.make_async_copy(mb[b2], agg_sh.at[db[b2]],
                                              ssem[b2]).wait()
                    fire_inputs(g + 2, b2)

        @pl.loop(0, nch // 4)
        def quad(p):
            for b in range(4):
                run_chunk(p * 4 + b, b, True)

        for gr in range(nch - (nch % 4), nch):
            run_chunk(gr, gr % 4, False)

        for gl in (nch - 4, nch - 3, nch - 2, nch - 1):
            b = gl % 4
            pltpu.make_async_copy(mb[b], agg_sh.at[db[b]], ssem[b]).wait()

        plsc.subcore_barrier()
        pltpu.sync_copy(agg_sh.at[pl.ds(s * rpt, rpt)],
                        agg_hbm.at[c, pl.ds(s * rpt, rpt)])

    dma = pltpu.SemaphoreType.DMA
    kern_c = pl.kernel(
        body_c,
        out_type=jax.ShapeDtypeStruct((NC, n_pad, H), f32),
        mesh=mesh,
        scratch_types=(
            [pltpu.VMEM((CH, H), f32)] * 4 +
            [pltpu.VMEM((CH,), jnp.int32)] * 4 +
            [pltpu.VMEM_SHARED((n_pad, H), f32)] +
            [dma] * 12),
        compiler_params=pltpu.CompilerParams(needs_layout_passes=False),
    )
    agg = kern_c(msg, dst2d, zero_init)

    def body_d(ne_ref, a0_ref, a1_ref, w1_ref, w2_ref, b_ref, out_ref):
        acc = jnp.dot(ne_ref[...], w1_ref[...], preferred_element_type=f32)
        acc = acc + jnp.dot(a0_ref[...] + a1_ref[...], w2_ref[...],
                            preferred_element_type=f32)
        out_ref[...] = acc + b_ref[...]

    nbn = n_pad // 1024
    row_spec = pl.BlockSpec((1024, H), lambda i: (i, 0))
    node_emb = pl.pallas_call(
        body_d,
        grid=(nbn,),
        in_specs=[row_spec, row_spec, row_spec,
                  pl.BlockSpec((H, H), lambda i: (0, 0)),
                  pl.BlockSpec((H, H), lambda i: (0, 0)),
                  pl.BlockSpec((1, H), lambda i: (0, 0))],
        out_specs=row_spec,
        out_shape=jax.ShapeDtypeStruct((n_pad, H), f32),
    )(ne, agg[0], agg[1], comb_w.astype(f32)[:H], comb_w.astype(f32)[H:],
      comb_b.astype(f32)[None, :])

    node_embedding = node_emb[:N]
    node_vec = jnp.zeros((N, 3, H), f32)
    edge_weight = ew3.reshape(e2)[:E]
    edge_attr_out = attr_out
    edge_vec = jnp.stack([enx3.reshape(e2)[:E],
                          eny3.reshape(e2)[:E],
                          enz3.reshape(e2)[:E]], axis=-1)
    return (node_embedding, node_vec, edge_index, edge_weight,
            edge_attr_out, edge_vec)
